# Initial kernel scaffold; baseline (speedup 1.0000x reference)
import jax
import jax.numpy as jnp
from jax.experimental import pallas as pl


def kernel(x, w, b, gamma, beta):
    raise NotImplementedError("write your pallas kernel here")


# speedup vs baseline: 17.5282x; 17.5282x over previous
import functools

import jax
import jax.numpy as jnp
from jax.experimental import pallas as pl
from jax.experimental.pallas import tpu as pltpu


def _build_patches(xb):
    ho = xb.shape[1] - 1
    parts = []
    for ah in (0, 1):
        sl = xb[:, ah:ah + ho, :]
        for aw in (0, 1):
            parts.append(sl if aw == 0 else jnp.roll(sl, -1, axis=2))
    p = jnp.concatenate(parts, axis=0)
    return p.reshape(p.shape[0], ho * 128)


def _stats_kernel(xph_ref, w_ref, stat_ref, *, ho, wo):
    i = pl.program_id(1)

    @pl.when(i == 0)
    def _():
        stat_ref[...] = jnp.zeros_like(stat_ref)

    p = _build_patches(xph_ref[0])
    y = jnp.dot(w_ref[...], p, preferred_element_type=jnp.float32)
    y = y.reshape(y.shape[0], ho, 128)
    lane = jax.lax.broadcasted_iota(jnp.int32, (1, 1, 128), 2)
    ym = jnp.where(lane < wo, y, 0.0)
    stat_ref[0, 0] += jnp.sum(ym, axis=1)
    stat_ref[0, 1] += jnp.sum(ym * ym, axis=1)


def _apply_kernel(xph_ref, w_ref, scale_ref, shift_ref, o_ref, *, ho, wo):
    p = _build_patches(xph_ref[0])
    y = jnp.dot(w_ref[...], p, preferred_element_type=jnp.float32)
    co = y.shape[0]
    y = y.reshape(co, ho, 128)
    sc = scale_ref[...].reshape(co, 1, 1)
    sh = shift_ref[...].reshape(co, 1, 1)
    z = jnp.maximum(y * sc + sh, 0.0)
    o_ref[0] = z[:, :, :wo]


def kernel(x, w, b, gamma, beta):
    del b
    eps = 1e-5
    N, C_in, H, W = x.shape
    C_out = w.shape[0]
    Ho, Wo = H // 2, W // 2
    Hp, Wp = Ho + 1, Wo + 1
    K = C_in * 16
    M = N * Ho * Wo

    xp = jnp.pad(x, ((0, 0), (0, 0), (1, 1), (1, 1)))
    ph = xp.reshape(N, C_in, Hp, 2, Wp, 2)
    ph = ph.transpose(0, 1, 3, 5, 2, 4).reshape(N, 4 * C_in, Hp, Wp)
    xph = jnp.pad(ph, ((0, 0), (0, 0), (0, 0), (0, 128 - Wp)))

    w6 = w.reshape(C_out, C_in, 2, 2, 2, 2)
    w2 = w6.transpose(0, 2, 4, 1, 3, 5).reshape(C_out, K)

    NCORE = 2
    per_core = N // NCORE
    vmem_limit = 64 << 20

    stats = pl.pallas_call(
        functools.partial(_stats_kernel, ho=Ho, wo=Wo),
        out_shape=jax.ShapeDtypeStruct((NCORE, 2, C_out, 128), jnp.float32),
        grid=(NCORE, per_core),
        in_specs=[
            pl.BlockSpec((1, 4 * C_in, Hp, 128),
                         lambda c, i, pc=per_core: (c * pc + i, 0, 0, 0)),
            pl.BlockSpec((C_out, K), lambda c, i: (0, 0)),
        ],
        out_specs=pl.BlockSpec((1, 2, C_out, 128), lambda c, i: (c, 0, 0, 0)),
        compiler_params=pltpu.CompilerParams(
            dimension_semantics=("parallel", "arbitrary"),
            vmem_limit_bytes=vmem_limit),
    )(xph, w2)

    st = jnp.sum(stats, axis=(0, 3))
    mean = st[0] / jnp.float32(M)
    var = st[1] / jnp.float32(M) - mean * mean
    scale = gamma.astype(jnp.float32) * jax.lax.rsqrt(var + jnp.float32(eps))
    shift = beta.astype(jnp.float32) - mean * scale

    out = pl.pallas_call(
        functools.partial(_apply_kernel, ho=Ho, wo=Wo),
        out_shape=jax.ShapeDtypeStruct((N, C_out, Ho, Wo), jnp.float32),
        grid=(N,),
        in_specs=[
            pl.BlockSpec((1, 4 * C_in, Hp, 128), lambda n: (n, 0, 0, 0)),
            pl.BlockSpec((C_out, K), lambda n: (0, 0)),
            pl.BlockSpec((C_out, 1), lambda n: (0, 0)),
            pl.BlockSpec((C_out, 1), lambda n: (0, 0)),
        ],
        out_specs=pl.BlockSpec((1, C_out, Ho, Wo), lambda n: (n, 0, 0, 0)),
        compiler_params=pltpu.CompilerParams(
            dimension_semantics=("parallel",),
            vmem_limit_bytes=vmem_limit),
    )(xph, w2, scale.reshape(C_out, 1), shift.reshape(C_out, 1))

    return out


# speedup vs baseline: 18.1207x; 1.0338x over previous
import functools

import jax
import jax.numpy as jnp
from jax.experimental import pallas as pl
from jax.experimental.pallas import tpu as pltpu


def _conv_block(xb, w):
    m = (xb.shape[1] // 128 - 2) * 128
    parts = [xb[:, ah * 128 + aw: ah * 128 + aw + m]
             for ah in (0, 1) for aw in (0, 1)]
    p = jnp.concatenate(parts, axis=0)
    return jnp.dot(w, p, preferred_element_type=jnp.float32)


def _stats_kernel(xph_ref, w_ref, stat_ref, *, ho, imgs):
    i = pl.program_id(1)

    @pl.when(i == 0)
    def _():
        stat_ref[...] = jnp.zeros_like(stat_ref)

    for img in range(imgs):
        y = _conv_block(xph_ref[img], w_ref[...])
        co = y.shape[0]
        y3 = y.reshape(co, ho, 128)
        stat_ref[0, 0] += jnp.sum(y3, axis=1)
        stat_ref[0, 1] += jnp.sum(y3 * y3, axis=1)


def _apply_kernel(xph_ref, w_ref, scale_ref, shift_ref, o_ref, *, ho, wo, imgs):
    sc = scale_ref[...]
    sh = shift_ref[...]
    for img in range(imgs):
        y = _conv_block(xph_ref[img], w_ref[...])
        co = y.shape[0]
        z = jnp.maximum(y * sc + sh, 0.0)
        o_ref[img] = z.reshape(co, ho, 128)[:, :, :wo]


def kernel(x, w, b, gamma, beta):
    del b
    eps = 1e-5
    N, C_in, H, W = x.shape
    C_out = w.shape[0]
    Ho, Wo = H // 2, W // 2
    Hp, Wp = Ho + 1, Wo + 1
    M = N * Ho * Wo

    xp = jnp.pad(x, ((0, 0), (0, 0), (1, 1), (1, 1)))
    ph = xp.reshape(N, C_in, Hp, 2, Wp, 2)
    ph = ph.transpose(0, 1, 3, 5, 2, 4).reshape(N, 4 * C_in, Hp, Wp)
    xph = jnp.pad(ph, ((0, 0), (0, 16 - 4 * C_in), (0, 1), (0, 128 - Wp)))
    xph = xph.reshape(N, 16, (Hp + 1) * 128)

    w6 = w.reshape(C_out, C_in, 2, 2, 2, 2)
    w2 = w6.transpose(0, 2, 4, 1, 3, 5).reshape(C_out, 2, 2, 4 * C_in)
    w2 = jnp.pad(w2, ((0, 0), (0, 0), (0, 0), (0, 16 - 4 * C_in)))
    w2 = w2.reshape(C_out, 64)

    NCORE = 2
    IMG_A = 4 if N % (NCORE * 4) == 0 else 1
    IMG_B = 2 if N % 2 == 0 else 1
    vmem_limit = 100 << 20

    stats = pl.pallas_call(
        functools.partial(_stats_kernel, ho=Ho, imgs=IMG_A),
        out_shape=jax.ShapeDtypeStruct((NCORE, 2, C_out, 128), jnp.float32),
        grid=(NCORE, N // (NCORE * IMG_A)),
        in_specs=[
            pl.BlockSpec((IMG_A, 16, (Hp + 1) * 128),
                         lambda c, i, nb=N // NCORE // IMG_A: (c * nb + i, 0, 0)),
            pl.BlockSpec((C_out, 64), lambda c, i: (0, 0)),
        ],
        out_specs=pl.BlockSpec((1, 2, C_out, 128), lambda c, i: (c, 0, 0, 0)),
        compiler_params=pltpu.CompilerParams(
            dimension_semantics=("parallel", "arbitrary"),
            vmem_limit_bytes=vmem_limit),
    )(xph, w2)

    st = jnp.sum(stats[:, :, :, :Wo], axis=(0, 3))
    ssum, ssq = st[0], st[1]
    mean = ssum / jnp.float32(M)
    var = ssq / jnp.float32(M) - mean * mean
    scale = gamma.astype(jnp.float32) * jax.lax.rsqrt(var + jnp.float32(eps))
    shift = beta.astype(jnp.float32) - mean * scale

    out = pl.pallas_call(
        functools.partial(_apply_kernel, ho=Ho, wo=Wo, imgs=IMG_B),
        out_shape=jax.ShapeDtypeStruct((N, C_out, Ho, Wo), jnp.float32),
        grid=(N // IMG_B,),
        in_specs=[
            pl.BlockSpec((IMG_B, 16, (Hp + 1) * 128), lambda n: (n, 0, 0)),
            pl.BlockSpec((C_out, 64), lambda n: (0, 0)),
            pl.BlockSpec((C_out, 1), lambda n: (0, 0)),
            pl.BlockSpec((C_out, 1), lambda n: (0, 0)),
        ],
        out_specs=pl.BlockSpec((IMG_B, C_out, Ho, Wo), lambda n: (n, 0, 0, 0)),
        compiler_params=pltpu.CompilerParams(
            dimension_semantics=("parallel",),
            vmem_limit_bytes=vmem_limit),
    )(xph, w2, scale.reshape(C_out, 1), shift.reshape(C_out, 1))

    return out
